# baseline (speedup 1.0000x reference)
import functools

import numpy as np
import jax
import jax.numpy as jnp
from jax import lax
from jax.experimental import pallas as pl
from jax.experimental.pallas import tpu as pltpu
from jax.experimental.pallas import tpu_sc as plsc

N_NODES = 100000
D = 128
D_OUT = 128
B = 20000
K = 25

NC = 2
NS = 16
NW = NC * NS
BP = 20480
C = 64
NCH = BP // (NW * C)
W_SUB = NCH * C

NBUF = 8
NPAIR = K // 2
KC = K * C


def _sc_body(feat, nodesp, neigh_t, xs_out, xsum_out,
             idxall, sidxall, planes, selfbuf, acc,
             sem_s, sem_p0, sem_p1, sem_p2, sem_p3,
             sem_p4, sem_p5, sem_p6, sem_p7, nch=NCH):
    c_idx = lax.axis_index("c")
    s_idx = lax.axis_index("s")
    psems = [sem_p0, sem_p1, sem_p2, sem_p3, sem_p4, sem_p5, sem_p6, sem_p7]
    wid = s_idx * NC + c_idx
    start = wid * (nch * C)

    pltpu.sync_copy(neigh_t.at[pl.ds(start * K, nch * KC)], idxall)
    pltpu.sync_copy(nodesp.at[pl.ds(start, nch * C)], sidxall)

    def chunk(i, carry):
        base = start + i * C
        ib = i * KC
        cp_self = pltpu.async_copy(
            feat.at[sidxall.at[pl.ds(i * C, C)]], selfbuf, sem_s)
        cps = [None] * NBUF
        for b in range(NBUF):
            cps[b] = pltpu.async_copy(
                feat.at[idxall.at[pl.ds(ib + b * C, C)]],
                planes.at[b], psems[b])
        for kp in range(NPAIR):
            ba = (2 * kp) % NBUF
            bb = (2 * kp + 1) % NBUF
            cps[ba].wait()
            cps[bb].wait()
            first = kp == 0

            @plsc.parallel_loop(0, C, unroll=2)
            def _(c, _ba=ba, _bb=bb, _first=first):
                for j in range(8):
                    sl = pl.ds(j * 16, 16)
                    v = planes[_ba, c, sl] + planes[_bb, c, sl]
                    if _first:
                        acc[c, sl] = v
                    else:
                        plsc.addupdate(acc.at[c, sl], v)

            for k in (2 * kp + NBUF, 2 * kp + NBUF + 1):
                if k < K:
                    cps[k % NBUF] = pltpu.async_copy(
                        feat.at[idxall.at[pl.ds(ib + k * C, C)]],
                        planes.at[k % NBUF], psems[k % NBUF])
        cps[(K - 1) % NBUF].wait()

        @plsc.parallel_loop(0, C, unroll=2)
        def _(c):
            for j in range(8):
                sl = pl.ds(j * 16, 16)
                plsc.addupdate(acc.at[c, sl], planes[(K - 1) % NBUF, c, sl])

        cp_self.wait()
        pltpu.sync_copy(acc, xsum_out.at[pl.ds(base, C)])
        pltpu.sync_copy(selfbuf, xs_out.at[pl.ds(base, C)])
        return carry

    lax.fori_loop(0, nch, chunk, 0)


def _sc_gather(feat, nodesp, neigh_t, n_rows):
    nch = n_rows // (NW * C)
    mesh = plsc.VectorSubcoreMesh(core_axis_name="c", subcore_axis_name="s")
    f = pl.kernel(
        functools.partial(_sc_body, nch=nch), mesh=mesh,
        out_type=(jax.ShapeDtypeStruct((n_rows, D), jnp.float32),
                  jax.ShapeDtypeStruct((n_rows, D), jnp.float32)),
        scratch_types=[
            pltpu.VMEM((nch * KC,), jnp.int32),
            pltpu.VMEM((nch * C,), jnp.int32),
            pltpu.VMEM((NBUF, C, D), jnp.float32),
            pltpu.VMEM((C, D), jnp.float32),
            pltpu.VMEM((C, D), jnp.float32),
        ] + [pltpu.SemaphoreType.DMA] * 9,
    )
    return f(feat, nodesp, neigh_t)


def _mm_body(xs_ref, xm_ref, ws_ref, wr_ref, o_ref):
    a = lax.dot_general(ws_ref[...], xs_ref[...],
                        (((1,), (1,)), ((), ())),
                        preferred_element_type=jnp.float32)
    b = lax.dot_general(wr_ref[...], xm_ref[...],
                        (((1,), (1,)), ((), ())),
                        preferred_element_type=jnp.float32)
    o_ref[...] = jnp.maximum(0.5 * a + (0.5 / K) * b, 0.0)


def _tc_combine(xs, xm, w_self, w_rel, nb):
    return pl.pallas_call(
        _mm_body,
        grid=(1,),
        in_specs=[
            pl.BlockSpec((nb, D), lambda i: (0, 0)),
            pl.BlockSpec((nb, D), lambda i: (0, 0)),
            pl.BlockSpec((D_OUT, D), lambda i: (0, 0)),
            pl.BlockSpec((D_OUT, D), lambda i: (0, 0)),
        ],
        out_specs=pl.BlockSpec((D_OUT, nb), lambda i: (0, 0)),
        out_shape=jax.ShapeDtypeStruct((D_OUT, nb), jnp.float32),
    )(xs, xm, w_self, w_rel)


BH = BP // 2


@jax.jit
def kernel(feat, nodes, neigh_idx, W_self, W_rel):
    pad_n = jnp.arange(B, BP, dtype=jnp.int32) % N_NODES
    pad_block = (jnp.arange((BP - B) * K, dtype=jnp.int32)
                 % N_NODES).reshape(BP - B, K)
    outs = []
    for h in range(2):
        lo = h * BH
        if h == 0:
            nodes_h = lax.slice(nodes, (lo,), (lo + BH,))
            neigh_h = lax.slice(neigh_idx, (lo, 0), (lo + BH, K))
        else:
            nodes_h = jnp.concatenate([nodes[lo:], pad_n])
            neigh_h = jnp.concatenate([neigh_idx[lo:], pad_block], axis=0)
        neigh_t = (neigh_h.reshape(BH // C, C, K)
                   .transpose(0, 2, 1).reshape(-1))
        xs, xsum = _sc_gather(feat, nodes_h, neigh_t, BH)
        nb = BH if h == 0 else B - BH
        outs.append(_tc_combine(xs, xsum, W_self, W_rel, nb))
    return jnp.concatenate(outs, axis=1)


# speedup vs baseline: 1.0260x; 1.0260x over previous
import functools

import numpy as np
import jax
import jax.numpy as jnp
from jax import lax
from jax.experimental import pallas as pl
from jax.experimental.pallas import tpu as pltpu
from jax.experimental.pallas import tpu_sc as plsc

N_NODES = 100000
D = 128
D_OUT = 128
B = 20000
K = 25

NC = 2
NS = 16
NW = NC * NS
BP = 20480
C = 64
NCH = BP // (NW * C)
W_SUB = NCH * C

NBUF = 8
NPAIR = K // 2
KC = K * C


def _sc_body(feat, nodesp, neigh_t, xs_out, xsum_out,
             idxall, sidxall, planes, selfbuf, acc,
             sem_s, sem_p0, sem_p1, sem_p2, sem_p3,
             sem_p4, sem_p5, sem_p6, sem_p7, nch=NCH):
    c_idx = lax.axis_index("c")
    s_idx = lax.axis_index("s")
    psems = [sem_p0, sem_p1, sem_p2, sem_p3, sem_p4, sem_p5, sem_p6, sem_p7]
    wid = s_idx * NC + c_idx
    start = wid * (nch * C)

    pltpu.sync_copy(neigh_t.at[pl.ds(start * K, nch * KC)], idxall)
    pltpu.sync_copy(nodesp.at[pl.ds(start, nch * C)], sidxall)

    def chunk(i, carry):
        base = start + i * C
        ib = i * KC
        cp_self = pltpu.async_copy(
            feat.at[sidxall.at[pl.ds(i * C, C)]], selfbuf, sem_s)
        cps = [None] * NBUF
        for b in range(NBUF):
            cps[b] = pltpu.async_copy(
                feat.at[idxall.at[pl.ds(ib + b * C, C)]],
                planes.at[b], psems[b])
        for kp in range(NPAIR):
            ba = (2 * kp) % NBUF
            bb = (2 * kp + 1) % NBUF
            cps[ba].wait()
            cps[bb].wait()
            first = kp == 0

            @plsc.parallel_loop(0, C, unroll=2)
            def _(c, _ba=ba, _bb=bb, _first=first):
                for j in range(8):
                    sl = pl.ds(j * 16, 16)
                    v = planes[_ba, c, sl] + planes[_bb, c, sl]
                    if _first:
                        acc[c, sl] = v
                    else:
                        plsc.addupdate(acc.at[c, sl], v)

            for k in (2 * kp + NBUF, 2 * kp + NBUF + 1):
                if k < K:
                    cps[k % NBUF] = pltpu.async_copy(
                        feat.at[idxall.at[pl.ds(ib + k * C, C)]],
                        planes.at[k % NBUF], psems[k % NBUF])
        cps[(K - 1) % NBUF].wait()

        @plsc.parallel_loop(0, C, unroll=2)
        def _(c):
            for j in range(8):
                sl = pl.ds(j * 16, 16)
                plsc.addupdate(acc.at[c, sl], planes[(K - 1) % NBUF, c, sl])

        cp_self.wait()
        pltpu.sync_copy(acc, xsum_out.at[pl.ds(base, C)])
        pltpu.sync_copy(selfbuf, xs_out.at[pl.ds(base, C)])
        return carry

    lax.fori_loop(0, nch, chunk, 0)


def _sc_gather(feat, nodesp, neigh_t, n_rows):
    nch = n_rows // (NW * C)
    mesh = plsc.VectorSubcoreMesh(core_axis_name="c", subcore_axis_name="s")
    f = pl.kernel(
        functools.partial(_sc_body, nch=nch), mesh=mesh,
        out_type=(jax.ShapeDtypeStruct((n_rows, D), jnp.float32),
                  jax.ShapeDtypeStruct((n_rows, D), jnp.float32)),
        scratch_types=[
            pltpu.VMEM((nch * KC,), jnp.int32),
            pltpu.VMEM((nch * C,), jnp.int32),
            pltpu.VMEM((NBUF, C, D), jnp.float32),
            pltpu.VMEM((C, D), jnp.float32),
            pltpu.VMEM((C, D), jnp.float32),
        ] + [pltpu.SemaphoreType.DMA] * 9,
    )
    return f(feat, nodesp, neigh_t)


def _mm_body(xs_ref, xm_ref, ws_ref, wr_ref, o_ref):
    a = lax.dot_general(ws_ref[...], xs_ref[...],
                        (((1,), (1,)), ((), ())),
                        preferred_element_type=jnp.float32)
    b = lax.dot_general(wr_ref[...], xm_ref[...],
                        (((1,), (1,)), ((), ())),
                        preferred_element_type=jnp.float32)
    o_ref[...] = jnp.maximum(0.5 * a + (0.5 / K) * b, 0.0)


def _tc_combine(xs, xm, w_self, w_rel, nb):
    return pl.pallas_call(
        _mm_body,
        grid=(1,),
        in_specs=[
            pl.BlockSpec((nb, D), lambda i: (0, 0)),
            pl.BlockSpec((nb, D), lambda i: (0, 0)),
            pl.BlockSpec((D_OUT, D), lambda i: (0, 0)),
            pl.BlockSpec((D_OUT, D), lambda i: (0, 0)),
        ],
        out_specs=pl.BlockSpec((D_OUT, nb), lambda i: (0, 0)),
        out_shape=jax.ShapeDtypeStruct((D_OUT, nb), jnp.float32),
    )(xs, xm, w_self, w_rel)


_PAD_N = (np.arange(B, BP) % N_NODES).astype(np.int32)
_PAD_BLOCK = (np.arange((BP - B) * K) % N_NODES).astype(np.int32) \
    .reshape(BP - B, K)


@jax.jit
def kernel(feat, nodes, neigh_idx, W_self, W_rel):
    nodesp = jnp.concatenate([nodes, jnp.asarray(_PAD_N)])
    neigh_t = (jnp.concatenate([neigh_idx, jnp.asarray(_PAD_BLOCK)], axis=0)
               .reshape(BP // C, C, K).transpose(0, 2, 1).reshape(-1))
    xs, xsum = _sc_gather(feat, nodesp, neigh_t, BP)
    return _tc_combine(xs, xsum, W_self, W_rel, B)


# speedup vs baseline: 1.0474x; 1.0209x over previous
import functools

import numpy as np
import jax
import jax.numpy as jnp
from jax import lax
from jax.experimental import pallas as pl
from jax.experimental.pallas import tpu as pltpu
from jax.experimental.pallas import tpu_sc as plsc

N_NODES = 100000
D = 128
D_OUT = 128
B = 20000
K = 25

NC = 2
NS = 16
NW = NC * NS
BP = 20480
C = 64
NCH = BP // (NW * C)
W_SUB = NCH * C

NBUF = 8
NPAIR = K // 2
KC = K * C


def _sc_body(feat, nodesp, neigh_t, xs_out, xsum_out,
             idxall, sidxall, planes, selfbuf, acc,
             sem_s0, sem_s1, sem_p0, sem_p1, sem_p2, sem_p3,
             sem_p4, sem_p5, sem_p6, sem_p7,
             sem_oa0, sem_oa1, sem_os0, sem_os1, nch=NCH):
    c_idx = lax.axis_index("c")
    s_idx = lax.axis_index("s")
    psems = [sem_p0, sem_p1, sem_p2, sem_p3, sem_p4, sem_p5, sem_p6, sem_p7]
    ssems = [sem_s0, sem_s1]
    oasems = [sem_oa0, sem_oa1]
    ossems = [sem_os0, sem_os1]
    wid = s_idx * NC + c_idx
    start = wid * (nch * C)

    pltpu.sync_copy(neigh_t.at[pl.ds(start * K, nch * KC)], idxall)
    pltpu.sync_copy(nodesp.at[pl.ds(start, nch * C)], sidxall)

    def chunk_pair(i2, carry):
        for sub in range(2):
            i = 2 * i2 + sub
            base = start + i * C
            ib = i * KC
            @pl.when(i2 > 0)
            def _(sub=sub, base=base):
                pltpu.make_async_copy(
                    acc.at[sub], xsum_out.at[pl.ds(base, C)],
                    oasems[sub]).wait()
                pltpu.make_async_copy(
                    selfbuf.at[sub], xs_out.at[pl.ds(base, C)],
                    ossems[sub]).wait()

            cp_self = pltpu.async_copy(
                feat.at[sidxall.at[pl.ds(i * C, C)]], selfbuf.at[sub],
                ssems[sub])
            cps = [None] * NBUF
            for b in range(NBUF):
                cps[b] = pltpu.async_copy(
                    feat.at[idxall.at[pl.ds(ib + b * C, C)]],
                    planes.at[b], psems[b])
            for kp in range(NPAIR):
                ba = (2 * kp) % NBUF
                bb = (2 * kp + 1) % NBUF
                cps[ba].wait()
                cps[bb].wait()
                first = kp == 0

                @plsc.parallel_loop(0, C, unroll=2)
                def _(c, _ba=ba, _bb=bb, _first=first, _sub=sub):
                    for j in range(8):
                        sl = pl.ds(j * 16, 16)
                        v = planes[_ba, c, sl] + planes[_bb, c, sl]
                        if _first:
                            acc[_sub, c, sl] = v
                        else:
                            plsc.addupdate(acc.at[_sub, c, sl], v)

                for k in (2 * kp + NBUF, 2 * kp + NBUF + 1):
                    if k < K:
                        cps[k % NBUF] = pltpu.async_copy(
                            feat.at[idxall.at[pl.ds(ib + k * C, C)]],
                            planes.at[k % NBUF], psems[k % NBUF])
            cps[(K - 1) % NBUF].wait()

            @plsc.parallel_loop(0, C, unroll=2)
            def _(c, _sub=sub):
                for j in range(8):
                    sl = pl.ds(j * 16, 16)
                    plsc.addupdate(acc.at[_sub, c, sl],
                                   planes[(K - 1) % NBUF, c, sl])

            cp_self.wait()
            pltpu.async_copy(acc.at[sub], xsum_out.at[pl.ds(base, C)],
                             oasems[sub])
            pltpu.async_copy(selfbuf.at[sub], xs_out.at[pl.ds(base, C)],
                             ossems[sub])
        return carry

    lax.fori_loop(0, nch // 2, chunk_pair, 0)
    tail_base = start + (nch - 2) * C
    for sub in range(2):
        pltpu.make_async_copy(
            acc.at[sub], xsum_out.at[pl.ds(tail_base + sub * C, C)],
            oasems[sub]).wait()
        pltpu.make_async_copy(
            selfbuf.at[sub], xs_out.at[pl.ds(tail_base + sub * C, C)],
            ossems[sub]).wait()


def _sc_gather(feat, nodesp, neigh_t, n_rows):
    nch = n_rows // (NW * C)
    mesh = plsc.VectorSubcoreMesh(core_axis_name="c", subcore_axis_name="s")
    f = pl.kernel(
        functools.partial(_sc_body, nch=nch), mesh=mesh,
        out_type=(jax.ShapeDtypeStruct((n_rows, D), jnp.float32),
                  jax.ShapeDtypeStruct((n_rows, D), jnp.float32)),
        scratch_types=[
            pltpu.VMEM((nch * KC,), jnp.int32),
            pltpu.VMEM((nch * C,), jnp.int32),
            pltpu.VMEM((NBUF, C, D), jnp.float32),
            pltpu.VMEM((2, C, D), jnp.float32),
            pltpu.VMEM((2, C, D), jnp.float32),
        ] + [pltpu.SemaphoreType.DMA] * 14,
    )
    return f(feat, nodesp, neigh_t)


def _mm_body(xs_ref, xm_ref, ws_ref, wr_ref, o_ref):
    a = lax.dot_general(ws_ref[...], xs_ref[...],
                        (((1,), (1,)), ((), ())),
                        preferred_element_type=jnp.float32)
    b = lax.dot_general(wr_ref[...], xm_ref[...],
                        (((1,), (1,)), ((), ())),
                        preferred_element_type=jnp.float32)
    o_ref[...] = jnp.maximum(0.5 * a + (0.5 / K) * b, 0.0)


def _tc_combine(xs, xm, w_self, w_rel, nb):
    return pl.pallas_call(
        _mm_body,
        grid=(1,),
        in_specs=[
            pl.BlockSpec((nb, D), lambda i: (0, 0)),
            pl.BlockSpec((nb, D), lambda i: (0, 0)),
            pl.BlockSpec((D_OUT, D), lambda i: (0, 0)),
            pl.BlockSpec((D_OUT, D), lambda i: (0, 0)),
        ],
        out_specs=pl.BlockSpec((D_OUT, nb), lambda i: (0, 0)),
        out_shape=jax.ShapeDtypeStruct((D_OUT, nb), jnp.float32),
    )(xs, xm, w_self, w_rel)


_PAD_N = (np.arange(B, BP) % N_NODES).astype(np.int32)
_PAD_BLOCK = (np.arange((BP - B) * K) % N_NODES).astype(np.int32) \
    .reshape(BP - B, K)


@jax.jit
def kernel(feat, nodes, neigh_idx, W_self, W_rel):
    nodesp = jnp.concatenate([nodes, jnp.asarray(_PAD_N)])
    neigh_t = (jnp.concatenate([neigh_idx, jnp.asarray(_PAD_BLOCK)], axis=0)
               .reshape(BP // C, C, K).transpose(0, 2, 1).reshape(-1))
    xs, xsum = _sc_gather(feat, nodesp, neigh_t, BP)
    return _tc_combine(xs, xsum, W_self, W_rel, B)
